# baseline (speedup 1.0000x reference)
import functools

import jax
import jax.numpy as jnp
from jax import lax
from jax.experimental import pallas as pl
from jax.experimental.pallas import tpu as pltpu
from jax.experimental.pallas import tpu_sc as plsc

NC = 2
NS = 16
NW = NC * NS
L = 16


def _mf_kernel(b_per_w, n_feats):
    mesh = plsc.VectorSubcoreMesh(core_axis_name="c", subcore_axis_name="s")
    B = b_per_w * NW
    lines_per_w = b_per_w // 128

    @functools.partial(
        pl.kernel,
        mesh=mesh,
        compiler_params=pltpu.CompilerParams(needs_layout_passes=False),
        out_type=jax.ShapeDtypeStruct((B,), jnp.float32),
        scratch_types=[
            pltpu.VMEM((n_feats, b_per_w), jnp.float32),
            pltpu.VMEM((n_feats, b_per_w), jnp.float32),
            pltpu.VMEM((b_per_w // 128, 128), jnp.float32),
            pltpu.VMEM((b_per_w // 128, 128), jnp.float32),
            pltpu.VMEM((b_per_w,), jnp.float32),
            pltpu.SemaphoreType.DMA,
        ],
    )
    def k(uvt_hbm, ivt_hbm, ub_hbm, ib_hbm, out_hbm,
          us_v, is_v, ub_v, ib_v, out_v, sem):
        wid = lax.axis_index("s") * NC + lax.axis_index("c")
        base = wid * b_per_w
        wsl = pl.ds(base, b_per_w)

        cps = [
            pltpu.async_copy(uvt_hbm.at[:, wsl], us_v, sem),
            pltpu.async_copy(ivt_hbm.at[:, wsl], is_v, sem),
            pltpu.async_copy(
                ub_hbm.at[pl.ds(wid * lines_per_w, lines_per_w), :],
                ub_v, sem),
            pltpu.async_copy(
                ib_hbm.at[pl.ds(wid * lines_per_w, lines_per_w), :],
                ib_v, sem),
        ]
        for c in cps:
            c.wait()

        def compute(g, _):
            gsl = pl.ds(g * L, L)
            lsl = pl.ds((g * L) % 128, L)
            acc = ub_v[(g * L) // 128, lsl] + ib_v[(g * L) // 128, lsl]
            for f in range(n_feats):
                acc = acc + us_v[f, gsl] * is_v[f, gsl]
            out_v[gsl] = acc
            return _

        lax.fori_loop(0, b_per_w // L, compute, None)
        pltpu.sync_copy(out_v, out_hbm.at[wsl])

    return k


def kernel(user, item, u_bias, i_bias, u_embed, i_embed):
    B = user.shape[0]
    n_feats = u_embed.shape[1]
    b_per_w = B // NW

    uvt = jnp.take(u_embed, user, axis=0).T
    ivt = jnp.take(i_embed, item, axis=0).T
    n_rows = u_bias.shape[0]
    bias_tbl = jnp.concatenate([u_bias, i_bias], axis=0)
    bidx = jnp.concatenate([user, item + n_rows])
    bv = jnp.take(bias_tbl, bidx, axis=0)
    bv = bv.reshape(2 * B // 128, 128)
    ub = bv[: B // 128]
    ib = bv[B // 128:]

    k = _mf_kernel(b_per_w, n_feats)
    return k(uvt, ivt, ub, ib)


# speedup vs baseline: 1.9862x; 1.9862x over previous
import functools

import jax
import jax.numpy as jnp
from jax import lax
from jax.experimental import pallas as pl
from jax.experimental.pallas import tpu as pltpu
from jax.experimental.pallas import tpu_sc as plsc

NC = 2
NS = 16
NW = NC * NS
L = 16

CHUNK = 128
LINE = 128
TAIL_LINES = 8


def _mf_kernel(b_per_w, n_feats, n_lines, tail_start):
    mesh = plsc.VectorSubcoreMesh(core_axis_name="c", subcore_axis_name="s")
    B = b_per_w * NW
    n_chunks = b_per_w // CHUNK

    @functools.partial(
        pl.kernel,
        mesh=mesh,
        compiler_params=pltpu.CompilerParams(needs_layout_passes=False),
        out_type=jax.ShapeDtypeStruct((B,), jnp.float32),
        scratch_types=[
            pltpu.VMEM((b_per_w,), jnp.int32),
            pltpu.VMEM((b_per_w,), jnp.int32),
            pltpu.VMEM((CHUNK,), jnp.int32),
            pltpu.VMEM((CHUNK,), jnp.int32),
            pltpu.VMEM((CHUNK, LINE), jnp.float32),
            pltpu.VMEM((CHUNK, LINE), jnp.float32),
            pltpu.VMEM((TAIL_LINES, LINE), jnp.float32),
            pltpu.VMEM((TAIL_LINES, LINE), jnp.float32),
            pltpu.VMEM((n_feats, b_per_w), jnp.float32),
            pltpu.VMEM((n_feats, b_per_w), jnp.float32),
            pltpu.VMEM((b_per_w,), jnp.float32),
            pltpu.SemaphoreType.DMA,
        ],
    )
    def k(uraw_hbm, iraw_hbm, ubl_hbm, ibl_hbm, ubt_hbm, ibt_hbm,
          uvt_hbm, ivt_hbm, out_hbm,
          uraw_v, iraw_v, ul_v, il_v, ubs_v, ibs_v, ubt_v, ibt_v,
          us_v, is_v, out_v, sem):
        wid = lax.axis_index("s") * NC + lax.axis_index("c")
        base = wid * b_per_w
        wsl = pl.ds(base, b_per_w)

        rows_cp = [
            pltpu.async_copy(uvt_hbm.at[:, wsl], us_v, sem),
            pltpu.async_copy(ivt_hbm.at[:, wsl], is_v, sem),
        ]
        pltpu.sync_copy(ubt_hbm, ubt_v)
        pltpu.sync_copy(ibt_hbm, ibt_v)
        pltpu.sync_copy(uraw_hbm.at[wsl], uraw_v)
        pltpu.sync_copy(iraw_hbm.at[wsl], iraw_v)

        lane = lax.broadcasted_iota(jnp.int32, (L,), 0)

        for j in range(n_chunks):
            def lines(g, _, _j=j):
                gsl = pl.ds(g * L, L)
                csl = pl.ds(_j * CHUNK + g * L, L)
                ul_v[gsl] = jnp.minimum(uraw_v[csl] >> 7, n_lines - 1)
                il_v[gsl] = jnp.minimum(iraw_v[csl] >> 7, n_lines - 1)
                return _

            lax.fori_loop(0, CHUNK // L, lines, None)

            cps = [
                pltpu.async_copy(ubl_hbm.at[ul_v], ubs_v, sem),
                pltpu.async_copy(ibl_hbm.at[il_v], ibs_v, sem),
            ]
            if j == 0:
                cps = cps + rows_cp
            for c in cps:
                c.wait()

            def compute(g, _, _j=j):
                gsl = pl.ds(g * L, L)
                bsl = pl.ds(_j * CHUNK + g * L, L)
                rows = g * L + lane
                ur = uraw_v[bsl]
                ir = iraw_v[bsl]
                ubv = plsc.load_gather(ubs_v, [rows, ur & (LINE - 1)])
                ibv = plsc.load_gather(ibs_v, [rows, ir & (LINE - 1)])
                ud = jnp.maximum(ur - tail_start, 0)
                idd = jnp.maximum(ir - tail_start, 0)
                ut = plsc.load_gather(ubt_v, [ud >> 7, ud & (LINE - 1)])
                it = plsc.load_gather(ibt_v, [idd >> 7, idd & (LINE - 1)])
                acc = (jnp.where(ur >= tail_start, ut, ubv)
                       + jnp.where(ir >= tail_start, it, ibv))
                for f in range(n_feats):
                    acc = acc + us_v[f, bsl] * is_v[f, bsl]
                out_v[bsl] = acc
                return _

            lax.fori_loop(0, CHUNK // L, compute, None)

        pltpu.sync_copy(out_v, out_hbm.at[wsl])

    return k


def kernel(user, item, u_bias, i_bias, u_embed, i_embed):
    B = user.shape[0]
    n_rows, n_feats = u_embed.shape
    b_per_w = B // NW

    user = user.astype(jnp.int32)
    item = item.astype(jnp.int32)

    uvt = jnp.take(u_embed, user, axis=0).T
    ivt = jnp.take(i_embed, item, axis=0).T

    tail_start = n_rows - TAIL_LINES * LINE
    n_lines = -(-(tail_start + 1) // LINE)
    n_lines = -(-n_lines // 8) * 8
    ubl = u_bias[:n_lines * LINE].reshape(n_lines, LINE)
    ibl = i_bias[:n_lines * LINE].reshape(n_lines, LINE)
    ubt = u_bias[tail_start:].reshape(TAIL_LINES, LINE)
    ibt = i_bias[tail_start:].reshape(TAIL_LINES, LINE)

    k = _mf_kernel(b_per_w, n_feats, n_lines, tail_start)
    return k(user, item, ubl, ibl, ubt, ibt, uvt, ivt)


# speedup vs baseline: 2.0097x; 1.0118x over previous
import functools

import jax
import jax.numpy as jnp
from jax import lax
from jax.experimental import pallas as pl
from jax.experimental.pallas import tpu as pltpu
from jax.experimental.pallas import tpu_sc as plsc

NC = 2
NS = 16
NW = NC * NS
L = 16

CHUNK = 128
LINE = 128
TAIL_LINES = 8


def _mf_kernel(b_per_w, n_feats, n_lines, tail_start):
    mesh = plsc.VectorSubcoreMesh(core_axis_name="c", subcore_axis_name="s")
    B = b_per_w * NW
    n_chunks = b_per_w // CHUNK

    @functools.partial(
        pl.kernel,
        mesh=mesh,
        compiler_params=pltpu.CompilerParams(needs_layout_passes=False),
        out_type=jax.ShapeDtypeStruct((B,), jnp.float32),
        scratch_types=[
            pltpu.VMEM((b_per_w,), jnp.int32),
            pltpu.VMEM((b_per_w,), jnp.int32),
            pltpu.VMEM((CHUNK,), jnp.int32),
            pltpu.VMEM((CHUNK,), jnp.int32),
            pltpu.VMEM((CHUNK,), jnp.int32),
            pltpu.VMEM((CHUNK,), jnp.int32),
            pltpu.VMEM((CHUNK, LINE), jnp.float32),
            pltpu.VMEM((CHUNK, LINE), jnp.float32),
            pltpu.VMEM((CHUNK, LINE), jnp.float32),
            pltpu.VMEM((CHUNK, LINE), jnp.float32),
            pltpu.VMEM((TAIL_LINES, LINE), jnp.float32),
            pltpu.VMEM((TAIL_LINES, LINE), jnp.float32),
            pltpu.VMEM((n_feats, b_per_w), jnp.float32),
            pltpu.VMEM((n_feats, b_per_w), jnp.float32),
            pltpu.VMEM((b_per_w,), jnp.float32),
            pltpu.SemaphoreType.DMA,
        ],
    )
    def k(uraw_hbm, iraw_hbm, ubl_hbm, ibl_hbm, ubt_hbm, ibt_hbm,
          uvt_hbm, ivt_hbm, out_hbm,
          uraw_v, iraw_v, ul0_v, ul1_v, il0_v, il1_v,
          ubs0_v, ubs1_v, ibs0_v, ibs1_v, ubt_v, ibt_v,
          us_v, is_v, out_v, sem):
        ul_b = (ul0_v, ul1_v)
        il_b = (il0_v, il1_v)
        ubs_b = (ubs0_v, ubs1_v)
        ibs_b = (ibs0_v, ibs1_v)
        wid = lax.axis_index("s") * NC + lax.axis_index("c")
        base = wid * b_per_w
        wsl = pl.ds(base, b_per_w)

        rows_cp = [
            pltpu.async_copy(uvt_hbm.at[:, wsl], us_v, sem),
            pltpu.async_copy(ivt_hbm.at[:, wsl], is_v, sem),
        ]
        pltpu.sync_copy(ubt_hbm, ubt_v)
        pltpu.sync_copy(ibt_hbm, ibt_v)
        pltpu.sync_copy(uraw_hbm.at[wsl], uraw_v)
        pltpu.sync_copy(iraw_hbm.at[wsl], iraw_v)

        lane = lax.broadcasted_iota(jnp.int32, (L,), 0)

        def build_lines(j, buf):
            def lines(g, _, _j=j, _buf=buf):
                gsl = pl.ds(g * L, L)
                csl = pl.ds(_j * CHUNK + g * L, L)
                ul_b[_buf][gsl] = jnp.minimum(uraw_v[csl] >> 7, n_lines - 1)
                il_b[_buf][gsl] = jnp.minimum(iraw_v[csl] >> 7, n_lines - 1)
                return _
            lax.fori_loop(0, CHUNK // L, lines, None)

        def fire(buf):
            return [
                pltpu.async_copy(ubl_hbm.at[ul_b[buf]], ubs_b[buf], sem),
                pltpu.async_copy(ibl_hbm.at[il_b[buf]], ibs_b[buf], sem),
            ]

        build_lines(0, 0)
        pending = fire(0) + rows_cp

        for j in range(n_chunks):
            buf = j % 2
            if j + 1 < n_chunks:
                build_lines(j + 1, 1 - buf)
            for c in pending:
                c.wait()
            if j + 1 < n_chunks:
                pending = fire(1 - buf)

            def compute(g, _, _j=j, _buf=buf):
                gsl = pl.ds(g * L, L)
                bsl = pl.ds(_j * CHUNK + g * L, L)
                rows = g * L + lane
                ur = uraw_v[bsl]
                ir = iraw_v[bsl]
                ubv = plsc.load_gather(ubs_b[_buf], [rows, ur & (LINE - 1)])
                ibv = plsc.load_gather(ibs_b[_buf], [rows, ir & (LINE - 1)])
                ud = jnp.maximum(ur - tail_start, 0)
                idd = jnp.maximum(ir - tail_start, 0)
                ut = plsc.load_gather(ubt_v, [ud >> 7, ud & (LINE - 1)])
                it = plsc.load_gather(ibt_v, [idd >> 7, idd & (LINE - 1)])
                acc = (jnp.where(ur >= tail_start, ut, ubv)
                       + jnp.where(ir >= tail_start, it, ibv))
                for f in range(n_feats):
                    acc = acc + us_v[f, bsl] * is_v[f, bsl]
                out_v[bsl] = acc
                return _

            lax.fori_loop(0, CHUNK // L, compute, None)

        pltpu.sync_copy(out_v, out_hbm.at[wsl])

    return k


def kernel(user, item, u_bias, i_bias, u_embed, i_embed):
    B = user.shape[0]
    n_rows, n_feats = u_embed.shape
    b_per_w = B // NW

    user = user.astype(jnp.int32)
    item = item.astype(jnp.int32)

    uvt = jnp.take(u_embed, user, axis=0).T
    ivt = jnp.take(i_embed, item, axis=0).T

    tail_start = n_rows - TAIL_LINES * LINE
    n_lines = -(-(tail_start + 1) // LINE)
    n_lines = -(-n_lines // 8) * 8
    ubl = u_bias[:n_lines * LINE].reshape(n_lines, LINE)
    ibl = i_bias[:n_lines * LINE].reshape(n_lines, LINE)
    ubt = u_bias[tail_start:].reshape(TAIL_LINES, LINE)
    ibt = i_bias[tail_start:].reshape(TAIL_LINES, LINE)

    k = _mf_kernel(b_per_w, n_feats, n_lines, tail_start)
    return k(user, item, ubl, ibl, ubt, ibt, uvt, ivt)
